# baseline (speedup 1.0000x reference)
import functools

import jax
import jax.numpy as jnp
from jax import lax
from jax.experimental import pallas as pl
from jax.experimental.pallas import tpu as pltpu
from jax.experimental.pallas import tpu_sc as plsc

B, N, M, D, K = 16, 4096, 64, 128, 512
NB = 64


def _scores_block(xb, des):
    accs = []
    for j in range(8):
        acc = jnp.zeros((NB, D), jnp.float32)
        for mt in range(8):
            m = 8 * mt + j
            diff = xb - des[m][None, :]
            acc = acc + diff * diff
        accs.append(acc)
    t = [accs[i] + accs[i + 4] for i in range(4)]
    t = [t[i] + t[i + 2] for i in range(2)]
    a = t[0] + t[1]
    return jnp.sum(a, axis=-1)


def _excl_cumsum_2d(mask_f32, upper_incl, lower_strict):
    rowcum = lax.dot_general(mask_f32, upper_incl, (((1,), (0,)), ((), ())),
                             preferred_element_type=jnp.float32)
    below = lax.dot_general(lower_strict, rowcum, (((1,), (0,)), ((), ())),
                            preferred_element_type=jnp.float32)
    off = below[:, -1:]
    return rowcum - mask_f32 + off


def _tc_body(x_ref, des_ref, idx_ref):
    x = x_ref[0]
    des = des_ref[0]

    sc = []
    for nb in range(N // NB):
        sc.append(_scores_block(x[nb * NB:(nb + 1) * NB], des))
    s_flat = jnp.concatenate(sc)

    k_flat = lax.bitcast_convert_type(s_flat, jnp.int32)

    lo = jnp.int32(0)
    hi = jnp.int32(0x7f800000)
    for _ in range(31):
        mid = lo + ((hi - lo) >> 1)
        cnt = jnp.sum((k_flat <= mid).astype(jnp.int32))
        take = cnt >= K
        hi = jnp.where(take, mid, hi)
        lo = jnp.where(take, lo, mid + 1)
    kT = lo

    R, C = N // 128, 128
    k2d = k_flat.reshape(R, C)
    strict2d = k2d < kT
    tie2d = k2d == kT
    upper_incl = (lax.broadcasted_iota(jnp.int32, (C, C), 0) <=
                  lax.broadcasted_iota(jnp.int32, (C, C), 1)
                  ).astype(jnp.float32)
    lower_strict = (lax.broadcasted_iota(jnp.int32, (R, R), 0) >
                    lax.broadcasted_iota(jnp.int32, (R, R), 1)
                    ).astype(jnp.float32)
    spos = _excl_cumsum_2d(strict2d.astype(jnp.float32),
                           upper_incl, lower_strict).astype(jnp.int32)
    tpos = _excl_cumsum_2d(tie2d.astype(jnp.float32),
                           upper_incl, lower_strict).astype(jnp.int32)
    S = jnp.sum(strict2d.astype(jnp.int32))
    posk2d = jnp.where(
        strict2d, spos,
        jnp.where(tie2d & (tpos < (K - S)), S + tpos, jnp.int32(1 << 30)))
    posk_flat = posk2d.reshape(N)

    PCH = 128
    posk_row = jnp.broadcast_to(posk_flat[None, :], (PCH, N))
    nid_row = lax.broadcasted_iota(jnp.int32, (PCH, N), 1)
    k_row = jnp.broadcast_to(k_flat[None, :], (PCH, N))
    cn_parts, ck_parts = [], []
    for pc in range(K // PCH):
        p_col = pc * PCH + lax.broadcasted_iota(jnp.int32, (PCH, N), 0)
        hit = posk_row == p_col
        cn_parts.append(jnp.sum(jnp.where(hit, nid_row, 0), axis=1))
        ck_parts.append(jnp.sum(jnp.where(hit, k_row, 0), axis=1))
    cn_flat = jnp.concatenate(cn_parts)
    ck_flat = jnp.concatenate(ck_parts)

    cn_col = cn_flat.reshape(K, 1)
    ck_col = ck_flat.reshape(K, 1)
    ckn_row = jnp.broadcast_to(ck_flat[None, :], (PCH, K))
    cnn_row = jnp.broadcast_to(cn_flat[None, :], (PCH, K))
    crank = jnp.zeros((K,), jnp.int32)
    for pc in range(K // PCH):
        p0 = pc * PCH
        kj = jnp.broadcast_to(ck_col[p0:p0 + PCH], (PCH, K))
        nj = jnp.broadcast_to(cn_col[p0:p0 + PCH], (PCH, K))
        dk = kj - ckn_row
        lt = -lax.shift_right_arithmetic(dk, 31)
        tlt = -lax.shift_right_arithmetic(nj - cnn_row, 31)
        crank = crank + jnp.sum(jnp.where(dk == 0, tlt, lt), axis=0)

    crank_row = jnp.broadcast_to(crank[None, :], (PCH, K))
    cols = []
    for kc in range(K // PCH):
        kid_col = kc * PCH + lax.broadcasted_iota(jnp.int32, (PCH, K), 0)
        hit = crank_row == kid_col
        cols.append(jnp.sum(jnp.where(hit, cnn_row, 0), axis=1,
                            keepdims=True))
    idx_ref[0] = jnp.concatenate(cols, axis=0) + pl.program_id(0) * N


def _tc_indices(x, desired):
    return pl.pallas_call(
        _tc_body,
        grid=(B,),
        in_specs=[
            pl.BlockSpec((1, N, D), lambda b: (b, 0, 0)),
            pl.BlockSpec((1, M, D), lambda b: (b, 0, 0)),
        ],
        out_specs=pl.BlockSpec((1, K, 1), lambda b: (b, 0, 0)),
        out_shape=jax.ShapeDtypeStruct((B, K, 1), jnp.int32),
        compiler_params=pltpu.CompilerParams(
            dimension_semantics=("parallel",)),
    )(x, desired)


_NC, _NS = 2, 16
_NW = _NC * _NS
_ROWS = B * K
_PER_W = _ROWS // _NW
_CH = 128
_NCH = _PER_W // _CH


@functools.lru_cache(maxsize=1)
def _make_sc_gather():
    @functools.partial(
        pl.kernel,
        mesh=plsc.VectorSubcoreMesh(core_axis_name="c", subcore_axis_name="s"),
        out_type=jax.ShapeDtypeStruct((_ROWS, D), jnp.float32),
        scratch_types=[
            pltpu.VMEM((_CH,), jnp.int32),
            pltpu.VMEM((_CH,), jnp.int32),
            pltpu.VMEM((_CH, D), jnp.float32),
            pltpu.VMEM((_CH, D), jnp.float32),
            pltpu.SemaphoreType.DMA,
        ],
    )
    def _sc_gather(table_hbm, idx_hbm, out_hbm,
                   idx_a, idx_b, rows_a, rows_b, sem):
        wid = lax.axis_index("s") * _NC + lax.axis_index("c")
        for g, (iv, rv) in enumerate(((idx_a, rows_a), (idx_b, rows_b))):
            base = wid * _PER_W + g * _CH
            pltpu.sync_copy(idx_hbm.at[pl.ds(base, _CH)], iv)
            pltpu.async_copy(table_hbm.at[iv], rv, sem).wait()
            pltpu.sync_copy(rv, out_hbm.at[pl.ds(base, _CH)])

    return _sc_gather


@jax.jit
def kernel(x, desired):
    idx = _tc_indices(x, desired)
    table = x.reshape(B * N, D)
    out = _make_sc_gather()(table, idx.reshape(_ROWS))
    return out.reshape(B, K, D)


# speedup vs baseline: 1.0002x; 1.0002x over previous
import functools

import jax
import jax.numpy as jnp
from jax import lax
from jax.experimental import pallas as pl
from jax.experimental.pallas import tpu as pltpu
from jax.experimental.pallas import tpu_sc as plsc

B, N, M, D, K = 16, 4096, 64, 128, 512
NB = 64


def _scores_block(xb, des):
    accs = []
    for j in range(8):
        acc = None
        for mt in range(8):
            m = 8 * mt + j
            diff = xb - des[m][None, :]
            sq = diff * diff
            acc = sq if acc is None else acc + sq
        accs.append(acc)
    t = [accs[i] + accs[i + 4] for i in range(4)]
    t = [t[i] + t[i + 2] for i in range(2)]
    a = t[0] + t[1]
    return jnp.sum(a, axis=-1)


def _excl_cumsum_2d(mask_f32, upper_incl, lower_strict):
    rowcum = lax.dot_general(mask_f32, upper_incl, (((1,), (0,)), ((), ())),
                             preferred_element_type=jnp.float32)
    below = lax.dot_general(lower_strict, rowcum, (((1,), (0,)), ((), ())),
                            preferred_element_type=jnp.float32)
    off = below[:, -1:]
    return rowcum - mask_f32 + off


def _tc_body(x_ref, des_ref, idx_ref):
    x = x_ref[0]
    des = des_ref[0]

    sc = []
    for nb in range(N // NB):
        sc.append(_scores_block(x[nb * NB:(nb + 1) * NB], des))
    s_flat = jnp.concatenate(sc)

    k_flat = lax.bitcast_convert_type(s_flat, jnp.int32)

    lo = jnp.int32(0)
    hi = jnp.int32(0x7f800000)
    for _ in range(31):
        mid = lo + ((hi - lo) >> 1)
        cnt = jnp.sum((k_flat <= mid).astype(jnp.int32))
        take = cnt >= K
        hi = jnp.where(take, mid, hi)
        lo = jnp.where(take, lo, mid + 1)
    kT = lo

    R, C = N // 128, 128
    k2d = k_flat.reshape(R, C)
    strict2d = k2d < kT
    tie2d = k2d == kT
    upper_incl = (lax.broadcasted_iota(jnp.int32, (C, C), 0) <=
                  lax.broadcasted_iota(jnp.int32, (C, C), 1)
                  ).astype(jnp.float32)
    lower_strict = (lax.broadcasted_iota(jnp.int32, (R, R), 0) >
                    lax.broadcasted_iota(jnp.int32, (R, R), 1)
                    ).astype(jnp.float32)
    spos = _excl_cumsum_2d(strict2d.astype(jnp.float32),
                           upper_incl, lower_strict).astype(jnp.int32)
    tpos = _excl_cumsum_2d(tie2d.astype(jnp.float32),
                           upper_incl, lower_strict).astype(jnp.int32)
    S = jnp.sum(strict2d.astype(jnp.int32))
    posk2d = jnp.where(
        strict2d, spos,
        jnp.where(tie2d & (tpos < (K - S)), S + tpos, jnp.int32(1 << 30)))
    posk_flat = posk2d.reshape(N)

    PCH = 128
    posk_row = jnp.broadcast_to(posk_flat[None, :], (PCH, N))
    nid_row = lax.broadcasted_iota(jnp.int32, (PCH, N), 1)
    k_row = jnp.broadcast_to(k_flat[None, :], (PCH, N))
    cn_parts, ck_parts = [], []
    for pc in range(K // PCH):
        p_col = pc * PCH + lax.broadcasted_iota(jnp.int32, (PCH, N), 0)
        hit = posk_row == p_col
        cn_parts.append(jnp.sum(jnp.where(hit, nid_row, 0), axis=1))
        ck_parts.append(jnp.sum(jnp.where(hit, k_row, 0), axis=1))
    cn_flat = jnp.concatenate(cn_parts)
    ck_flat = jnp.concatenate(ck_parts)

    cn_col = cn_flat.reshape(K, 1)
    ck_col = ck_flat.reshape(K, 1)
    ckn_row = jnp.broadcast_to(ck_flat[None, :], (PCH, K))
    cnn_row = jnp.broadcast_to(cn_flat[None, :], (PCH, K))
    crank = jnp.zeros((K,), jnp.int32)
    for pc in range(K // PCH):
        p0 = pc * PCH
        kj = jnp.broadcast_to(ck_col[p0:p0 + PCH], (PCH, K))
        nj = jnp.broadcast_to(cn_col[p0:p0 + PCH], (PCH, K))
        dk = kj - ckn_row
        lt = -lax.shift_right_arithmetic(dk, 31)
        tlt = -lax.shift_right_arithmetic(nj - cnn_row, 31)
        crank = crank + jnp.sum(jnp.where(dk == 0, tlt, lt), axis=0)

    crank_row = jnp.broadcast_to(crank[None, :], (PCH, K))
    cols = []
    for kc in range(K // PCH):
        kid_col = kc * PCH + lax.broadcasted_iota(jnp.int32, (PCH, K), 0)
        hit = crank_row == kid_col
        cols.append(jnp.sum(jnp.where(hit, cnn_row, 0), axis=1,
                            keepdims=True))
    idx_ref[0] = jnp.concatenate(cols, axis=0) + pl.program_id(0) * N


def _tc_indices(x, desired):
    return pl.pallas_call(
        _tc_body,
        grid=(B,),
        in_specs=[
            pl.BlockSpec((1, N, D), lambda b: (b, 0, 0)),
            pl.BlockSpec((1, M, D), lambda b: (b, 0, 0)),
        ],
        out_specs=pl.BlockSpec((1, K, 1), lambda b: (b, 0, 0)),
        out_shape=jax.ShapeDtypeStruct((B, K, 1), jnp.int32),
        compiler_params=pltpu.CompilerParams(
            dimension_semantics=("parallel",)),
    )(x, desired)


_NC, _NS = 2, 16
_NW = _NC * _NS
_ROWS = B * K
_PER_W = _ROWS // _NW
_CH = 128
_NCH = _PER_W // _CH


@functools.lru_cache(maxsize=1)
def _make_sc_gather():
    @functools.partial(
        pl.kernel,
        mesh=plsc.VectorSubcoreMesh(core_axis_name="c", subcore_axis_name="s"),
        out_type=jax.ShapeDtypeStruct((_ROWS, D), jnp.float32),
        scratch_types=[
            pltpu.VMEM((_CH,), jnp.int32),
            pltpu.VMEM((_CH,), jnp.int32),
            pltpu.VMEM((_CH, D), jnp.float32),
            pltpu.VMEM((_CH, D), jnp.float32),
            pltpu.SemaphoreType.DMA,
        ],
    )
    def _sc_gather(table_hbm, idx_hbm, out_hbm,
                   idx_a, idx_b, rows_a, rows_b, sem):
        wid = lax.axis_index("s") * _NC + lax.axis_index("c")
        for g, (iv, rv) in enumerate(((idx_a, rows_a), (idx_b, rows_b))):
            base = wid * _PER_W + g * _CH
            pltpu.sync_copy(idx_hbm.at[pl.ds(base, _CH)], iv)
            pltpu.async_copy(table_hbm.at[iv], rv, sem).wait()
            pltpu.sync_copy(rv, out_hbm.at[pl.ds(base, _CH)])

    return _sc_gather


@jax.jit
def kernel(x, desired):
    idx = _tc_indices(x, desired)
    table = x.reshape(B * N, D)
    out = _make_sc_gather()(table, idx.reshape(_ROWS))
    return out.reshape(B, K, D)


# speedup vs baseline: 1.0059x; 1.0056x over previous
import functools

import jax
import jax.numpy as jnp
from jax import lax
from jax.experimental import pallas as pl
from jax.experimental.pallas import tpu as pltpu
from jax.experimental.pallas import tpu_sc as plsc

B, N, M, D, K = 16, 4096, 64, 128, 512
NB = 64


def _scores_block(xb, des):
    def group(j):
        acc = None
        for mt in range(8):
            m = 8 * mt + j
            diff = xb - des[m][None, :]
            sq = diff * diff
            acc = sq if acc is None else acc + sq
        return acc
    t = [group(i) + group(i + 4) for i in range(4)]
    t = [t[i] + t[i + 2] for i in range(2)]
    a = t[0] + t[1]
    return jnp.sum(a, axis=-1)


def _excl_cumsum_2d(mask_f32, upper_incl, lower_strict):
    rowcum = lax.dot_general(mask_f32, upper_incl, (((1,), (0,)), ((), ())),
                             preferred_element_type=jnp.float32)
    below = lax.dot_general(lower_strict, rowcum, (((1,), (0,)), ((), ())),
                            preferred_element_type=jnp.float32)
    off = below[:, -1:]
    return rowcum - mask_f32 + off


def _tc_body(x_ref, des_ref, idx_ref):
    x = x_ref[0]
    des = des_ref[0]

    sc = []
    for nb in range(N // NB):
        sc.append(_scores_block(x[nb * NB:(nb + 1) * NB], des))
    s_flat = jnp.concatenate(sc)

    k_flat = lax.bitcast_convert_type(s_flat, jnp.int32)

    lo = jnp.int32(0)
    hi = jnp.int32(0x7f800000)
    for _ in range(31):
        mid = lo + ((hi - lo) >> 1)
        cnt = jnp.sum((k_flat <= mid).astype(jnp.int32))
        take = cnt >= K
        hi = jnp.where(take, mid, hi)
        lo = jnp.where(take, lo, mid + 1)
    kT = lo

    R, C = N // 128, 128
    k2d = k_flat.reshape(R, C)
    strict2d = k2d < kT
    tie2d = k2d == kT
    upper_incl = (lax.broadcasted_iota(jnp.int32, (C, C), 0) <=
                  lax.broadcasted_iota(jnp.int32, (C, C), 1)
                  ).astype(jnp.float32)
    lower_strict = (lax.broadcasted_iota(jnp.int32, (R, R), 0) >
                    lax.broadcasted_iota(jnp.int32, (R, R), 1)
                    ).astype(jnp.float32)
    spos = _excl_cumsum_2d(strict2d.astype(jnp.float32),
                           upper_incl, lower_strict).astype(jnp.int32)
    tpos = _excl_cumsum_2d(tie2d.astype(jnp.float32),
                           upper_incl, lower_strict).astype(jnp.int32)
    S = jnp.sum(strict2d.astype(jnp.int32))
    posk2d = jnp.where(
        strict2d, spos,
        jnp.where(tie2d & (tpos < (K - S)), S + tpos, jnp.int32(1 << 30)))
    posk_flat = posk2d.reshape(N)

    PCH = 128
    posk_row = jnp.broadcast_to(posk_flat[None, :], (PCH, N))
    nid_row = lax.broadcasted_iota(jnp.int32, (PCH, N), 1)
    k_row = jnp.broadcast_to(k_flat[None, :], (PCH, N))
    cn_parts, ck_parts = [], []
    for pc in range(K // PCH):
        p_col = pc * PCH + lax.broadcasted_iota(jnp.int32, (PCH, N), 0)
        hit = posk_row == p_col
        cn_parts.append(jnp.sum(jnp.where(hit, nid_row, 0), axis=1))
        ck_parts.append(jnp.sum(jnp.where(hit, k_row, 0), axis=1))
    cn_flat = jnp.concatenate(cn_parts)
    ck_flat = jnp.concatenate(ck_parts)

    cn_col = cn_flat.reshape(K, 1)
    ck_col = ck_flat.reshape(K, 1)
    ckn_row = jnp.broadcast_to(ck_flat[None, :], (PCH, K))
    cnn_row = jnp.broadcast_to(cn_flat[None, :], (PCH, K))
    crank = jnp.zeros((K,), jnp.int32)
    for pc in range(K // PCH):
        p0 = pc * PCH
        kj = jnp.broadcast_to(ck_col[p0:p0 + PCH], (PCH, K))
        nj = jnp.broadcast_to(cn_col[p0:p0 + PCH], (PCH, K))
        dk = kj - ckn_row
        lt = -lax.shift_right_arithmetic(dk, 31)
        tlt = -lax.shift_right_arithmetic(nj - cnn_row, 31)
        crank = crank + jnp.sum(jnp.where(dk == 0, tlt, lt), axis=0)

    crank_row = jnp.broadcast_to(crank[None, :], (PCH, K))
    cols = []
    for kc in range(K // PCH):
        kid_col = kc * PCH + lax.broadcasted_iota(jnp.int32, (PCH, K), 0)
        hit = crank_row == kid_col
        cols.append(jnp.sum(jnp.where(hit, cnn_row, 0), axis=1,
                            keepdims=True))
    idx_ref[0] = jnp.concatenate(cols, axis=0) + pl.program_id(0) * N


def _tc_indices(x, desired):
    return pl.pallas_call(
        _tc_body,
        grid=(B,),
        in_specs=[
            pl.BlockSpec((1, N, D), lambda b: (b, 0, 0)),
            pl.BlockSpec((1, M, D), lambda b: (b, 0, 0)),
        ],
        out_specs=pl.BlockSpec((1, K, 1), lambda b: (b, 0, 0)),
        out_shape=jax.ShapeDtypeStruct((B, K, 1), jnp.int32),
        compiler_params=pltpu.CompilerParams(
            dimension_semantics=("parallel",)),
    )(x, desired)


_NC, _NS = 2, 16
_NW = _NC * _NS
_ROWS = B * K
_PER_W = _ROWS // _NW
_CH = 128
_NCH = _PER_W // _CH


@functools.lru_cache(maxsize=1)
def _make_sc_gather():
    @functools.partial(
        pl.kernel,
        mesh=plsc.VectorSubcoreMesh(core_axis_name="c", subcore_axis_name="s"),
        out_type=jax.ShapeDtypeStruct((_ROWS, D), jnp.float32),
        scratch_types=[
            pltpu.VMEM((_CH,), jnp.int32),
            pltpu.VMEM((_CH,), jnp.int32),
            pltpu.VMEM((_CH, D), jnp.float32),
            pltpu.VMEM((_CH, D), jnp.float32),
            pltpu.SemaphoreType.DMA,
        ],
    )
    def _sc_gather(table_hbm, idx_hbm, out_hbm,
                   idx_a, idx_b, rows_a, rows_b, sem):
        wid = lax.axis_index("s") * _NC + lax.axis_index("c")
        for g, (iv, rv) in enumerate(((idx_a, rows_a), (idx_b, rows_b))):
            base = wid * _PER_W + g * _CH
            pltpu.sync_copy(idx_hbm.at[pl.ds(base, _CH)], iv)
            pltpu.async_copy(table_hbm.at[iv], rv, sem).wait()
            pltpu.sync_copy(rv, out_hbm.at[pl.ds(base, _CH)])

    return _sc_gather


@jax.jit
def kernel(x, desired):
    idx = _tc_indices(x, desired)
    table = x.reshape(B * N, D)
    out = _make_sc_gather()(table, idx.reshape(_ROWS))
    return out.reshape(B, K, D)
